# baseline (speedup 1.0000x reference)
import functools
import math

import jax
import jax.numpy as jnp
from jax import lax
from jax.experimental import pallas as pl
from jax.experimental.pallas import tpu as pltpu
from jax.experimental.pallas import tpu_sc as plsc

_C, _W, _H = 80, 128, 128
_N = 1000
_NPAD = 1024
_NTILES = 32
_R = 4
_MAXC = 3
_L = 16
_NBLK = _NPAD // _L

_GX = [math.exp(-float((r - _R) ** 2)) for r in range(2 * _R + 1)]


def _splat_kernel(
    heat_hbm, boxes_hbm, classes_hbm, out_hbm, buf, boxes_v, cls_v,
    sem_slab, sem_boxes,
):
    wid = lax.axis_index("s") * 2 + lax.axis_index("c")
    c0 = (wid * _C) // _NTILES
    c1 = ((wid + 1) * _C) // _NTILES

    slab_cp = pltpu.make_async_copy(heat_hbm.at[pl.ds(c0, _MAXC)], buf, sem_slab)
    slab_cp.start()
    boxes_cp = pltpu.make_async_copy(boxes_hbm, boxes_v, sem_boxes)
    boxes_cp.start()
    pltpu.sync_copy(classes_hbm, cls_v)
    boxes_cp.wait()
    slab_cp.wait()

    lanes = lax.iota(jnp.int32, _L)

    def splat(i, lc):
        bv = boxes_v[pl.ds(4 * i, _L)]
        cx = (bv[0] + bv[2]) // 2
        cy = (bv[1] + bv[3]) // 2
        y0 = jnp.clip(cy - _R, 0, _H - _L)
        dy = (lanes + (y0 - cy)).astype(jnp.float32)
        ey = jnp.exp(-(dy * dy))
        for r in range(2 * _R + 1):
            x = jnp.clip(cx + (r - _R), 0, _W - 1)
            g = ey if r == _R else _GX[r] * ey
            old = buf[lc, x, pl.ds(y0, _L)]
            buf[lc, x, pl.ds(y0, _L)] = jnp.maximum(old, g)

    def blk_body(b, carry):
        cls16 = cls_v[pl.ds(b * _L, _L)]
        own = jnp.logical_and(cls16 >= c0, cls16 < c1)

        @pl.when(jnp.any(own))
        def _():
            for j in range(_L):
                c = cls16[j]

                @pl.when(jnp.logical_and(c >= c0, c < c1))
                def _():
                    splat(b * _L + j, c - c0)

        return carry

    lax.fori_loop(0, _NBLK, blk_body, 0)

    pltpu.sync_copy(buf.at[pl.ds(0, 2)], out_hbm.at[pl.ds(c0, 2)])

    @pl.when(c1 - c0 == 3)
    def _():
        pltpu.sync_copy(buf.at[2], out_hbm.at[c0 + 2])


def kernel(heatmap, boxes, classes):
    boxes = boxes.astype(jnp.int32)
    classes = classes.astype(jnp.int32)
    boxes_flat = jnp.zeros((4 * _NPAD,), jnp.int32).at[: 4 * _N].set(
        boxes.reshape(-1)
    )
    cls_pad = jnp.full((_NPAD,), _C + 1, jnp.int32).at[:_N].set(classes)

    mesh = plsc.VectorSubcoreMesh(
        core_axis_name="c", subcore_axis_name="s", num_cores=2, num_subcores=16
    )
    run = functools.partial(
        pl.kernel,
        out_type=jax.ShapeDtypeStruct((_C, _W, _H), jnp.float32),
        mesh=mesh,
        compiler_params=pltpu.CompilerParams(needs_layout_passes=False),
        scratch_types=[
            pltpu.VMEM((_MAXC, _W, _H), jnp.float32),
            pltpu.VMEM((4 * _NPAD,), jnp.int32),
            pltpu.VMEM((_NPAD,), jnp.int32),
            pltpu.SemaphoreType.DMA,
            pltpu.SemaphoreType.DMA,
        ],
    )(_splat_kernel)
    return run(heatmap, boxes_flat, cls_pad)


# speedup vs baseline: 1.7566x; 1.7566x over previous
import functools
import math

import jax
import jax.numpy as jnp
from jax import lax
from jax.experimental import pallas as pl
from jax.experimental.pallas import tpu as pltpu
from jax.experimental.pallas import tpu_sc as plsc

_C, _W, _H = 80, 128, 128
_N = 1000
_NPAD = 1024
_NTILES = 32
_R = 4
_MAXC = 3
_L = 16
_NBLK = _NPAD // _L

_GX = [math.exp(-float((r - _R) ** 2)) for r in range(2 * _R + 1)]


def _splat_kernel(
    heat_hbm, boxes_hbm, classes_hbm, out_hbm, buf, boxes_v, cls_v,
    sem_slab, sem_boxes,
):
    wid = lax.axis_index("s") * 2 + lax.axis_index("c")
    c0 = (wid * _C) // _NTILES
    c1 = ((wid + 1) * _C) // _NTILES

    slab_cp = pltpu.make_async_copy(heat_hbm.at[pl.ds(c0, _MAXC)], buf, sem_slab)
    slab_cp.start()
    boxes_cp = pltpu.make_async_copy(boxes_hbm, boxes_v, sem_boxes)
    boxes_cp.start()
    pltpu.sync_copy(classes_hbm, cls_v)
    boxes_cp.wait()
    slab_cp.wait()

    lanes = lax.iota(jnp.int32, _L)

    def splat(i, lc):
        bv = boxes_v[pl.ds(4 * i, _L)]
        cx = (bv[0] + bv[2]) // 2
        cy = (bv[1] + bv[3]) // 2
        y0 = jnp.clip(cy - _R, 0, _H - _L)
        dy = (lanes + (y0 - cy)).astype(jnp.float32)
        ey = jnp.exp(-(dy * dy))
        for r in range(2 * _R + 1):
            x = jnp.clip(cx + (r - _R), 0, _W - 1)
            g = ey if r == _R else _GX[r] * ey
            old = buf[lc, x, pl.ds(y0, _L)]
            buf[lc, x, pl.ds(y0, _L)] = jnp.maximum(old, g)

    def body(i, carry):
        c = cls_v[pl.ds(i, _L)][0]

        @pl.when(jnp.logical_and(c >= c0, c < c1))
        def _():
            splat(i, c - c0)

        return carry

    lax.fori_loop(0, _N, body, 0)

    pltpu.sync_copy(buf.at[pl.ds(0, 2)], out_hbm.at[pl.ds(c0, 2)])

    @pl.when(c1 - c0 == 3)
    def _():
        pltpu.sync_copy(buf.at[2], out_hbm.at[c0 + 2])


def kernel(heatmap, boxes, classes):
    boxes = boxes.astype(jnp.int32)
    classes = classes.astype(jnp.int32)
    boxes_flat = jnp.zeros((4 * _NPAD,), jnp.int32).at[: 4 * _N].set(
        boxes.reshape(-1)
    )
    cls_pad = jnp.full((_NPAD,), _C + 1, jnp.int32).at[:_N].set(classes)

    mesh = plsc.VectorSubcoreMesh(
        core_axis_name="c", subcore_axis_name="s", num_cores=2, num_subcores=16
    )
    run = functools.partial(
        pl.kernel,
        out_type=jax.ShapeDtypeStruct((_C, _W, _H), jnp.float32),
        mesh=mesh,
        scratch_types=[
            pltpu.VMEM((_MAXC, _W, _H), jnp.float32),
            pltpu.VMEM((4 * _NPAD,), jnp.int32),
            pltpu.VMEM((_NPAD,), jnp.int32),
            pltpu.SemaphoreType.DMA,
            pltpu.SemaphoreType.DMA,
        ],
    )(_splat_kernel)
    return run(heatmap, boxes_flat, cls_pad)


# speedup vs baseline: 2.8193x; 1.6050x over previous
import functools
import math

import jax
import jax.numpy as jnp
from jax import lax
from jax.experimental import pallas as pl
from jax.experimental.pallas import tpu as pltpu
from jax.experimental.pallas import tpu_sc as plsc

_C, _W, _H = 80, 128, 128
_N = 1000
_NPAD = 1024
_NTILES = 32
_R = 4
_MAXC = 3
_L = 16
_NBLK = _NPAD // _L

_GX = [math.exp(-float((r - _R) ** 2)) for r in range(2 * _R + 1)]


def _splat_kernel(
    heat_hbm, boxes_hbm, classes_hbm, out_hbm, buf, boxes_v, cls_v,
    sem_slab, sem_boxes,
):
    wid = lax.axis_index("s") * 2 + lax.axis_index("c")
    c0 = (wid * _C) // _NTILES
    c1 = ((wid + 1) * _C) // _NTILES

    slab_cp = pltpu.make_async_copy(heat_hbm.at[pl.ds(c0, _MAXC)], buf, sem_slab)
    slab_cp.start()
    boxes_cp = pltpu.make_async_copy(boxes_hbm, boxes_v, sem_boxes)
    boxes_cp.start()
    pltpu.sync_copy(classes_hbm, cls_v)
    boxes_cp.wait()
    slab_cp.wait()

    lanes = lax.iota(jnp.int32, _L)

    def splat(i, lc):
        bv = boxes_v[pl.ds(4 * i, _L)]
        cx = (bv[0] + bv[2]) // 2
        cy = (bv[1] + bv[3]) // 2
        y0 = jnp.clip(cy - _R, 0, _H - _L)
        dy = (lanes + (y0 - cy)).astype(jnp.float32)
        ey = jnp.exp(-(dy * dy))
        for r in range(2 * _R + 1):
            x = jnp.clip(cx + (r - _R), 0, _W - 1)
            g = ey if r == _R else _GX[r] * ey
            old = buf[lc, x, pl.ds(y0, _L)]
            buf[lc, x, pl.ds(y0, _L)] = jnp.maximum(old, g)

    def treemax(m):
        for sh in (1, 2, 4, 8):
            m = jnp.maximum(m, m.at[lanes ^ sh].get(mode="promise_in_bounds"))
        return m[0]

    def blk_body(b, carry):
        base = b * _L
        cls16 = cls_v[pl.ds(base, _L)]
        own = jnp.logical_and(cls16 >= c0, cls16 < c1)
        mv = jnp.where(own, lanes + 1, 0)

        def w_cond(st):
            return st[1] > 0

        def w_body(st):
            mv, mx = st
            j = mx - 1
            c = cls16.at[jnp.zeros((_L,), jnp.int32) + j].get(
                mode="promise_in_bounds"
            )[0]
            splat(base + j, c - c0)
            mv = jnp.where(lanes == j, 0, mv)
            return mv, treemax(mv)

        lax.while_loop(w_cond, w_body, (mv, treemax(mv)))
        return carry

    lax.fori_loop(0, _NBLK, blk_body, 0)

    pltpu.sync_copy(buf.at[pl.ds(0, 2)], out_hbm.at[pl.ds(c0, 2)])

    @pl.when(c1 - c0 == 3)
    def _():
        pltpu.sync_copy(buf.at[2], out_hbm.at[c0 + 2])


def kernel(heatmap, boxes, classes):
    boxes = boxes.astype(jnp.int32)
    classes = classes.astype(jnp.int32)
    boxes_flat = jnp.zeros((4 * _NPAD,), jnp.int32).at[: 4 * _N].set(
        boxes.reshape(-1)
    )
    cls_pad = jnp.full((_NPAD,), _C + 1, jnp.int32).at[:_N].set(classes)

    mesh = plsc.VectorSubcoreMesh(
        core_axis_name="c", subcore_axis_name="s", num_cores=2, num_subcores=16
    )
    run = functools.partial(
        pl.kernel,
        out_type=jax.ShapeDtypeStruct((_C, _W, _H), jnp.float32),
        mesh=mesh,
        compiler_params=pltpu.CompilerParams(needs_layout_passes=False),
        scratch_types=[
            pltpu.VMEM((_MAXC, _W, _H), jnp.float32),
            pltpu.VMEM((4 * _NPAD,), jnp.int32),
            pltpu.VMEM((_NPAD,), jnp.int32),
            pltpu.SemaphoreType.DMA,
            pltpu.SemaphoreType.DMA,
        ],
    )(_splat_kernel)
    return run(heatmap, boxes_flat, cls_pad)


# speedup vs baseline: 2.8861x; 1.0237x over previous
import functools
import math

import jax
import jax.numpy as jnp
from jax import lax
from jax.experimental import pallas as pl
from jax.experimental.pallas import tpu as pltpu
from jax.experimental.pallas import tpu_sc as plsc

_C, _W, _H = 80, 128, 128
_N = 1000
_NPAD = 1024
_NTILES = 32
_R = 4
_MAXC = 3
_L = 16
_NBLK = _NPAD // _L

_GX = [math.exp(-float((r - _R) ** 2)) for r in range(2 * _R + 1)]


def _splat_kernel(
    heat_hbm, boxes_hbm, classes_hbm, out_hbm, buf, boxes_v, cls_v,
    sem_slab, sem_boxes,
):
    wid = lax.axis_index("s") * 2 + lax.axis_index("c")
    c0 = (wid * _C) // _NTILES
    c1 = ((wid + 1) * _C) // _NTILES

    slab_cp = pltpu.make_async_copy(heat_hbm.at[pl.ds(c0, _MAXC)], buf, sem_slab)
    slab_cp.start()
    boxes_cp = pltpu.make_async_copy(
        boxes_hbm, boxes_v.at[pl.ds(0, 4 * _N)], sem_boxes
    )
    boxes_cp.start()
    sent = jnp.full((_L,), _C + 1, jnp.int32)
    cls_v[pl.ds(_N - 8, _L)] = sent
    cls_v[pl.ds(_N + 8, _L)] = sent
    pltpu.sync_copy(classes_hbm, cls_v.at[pl.ds(0, _N)])
    boxes_cp.wait()
    slab_cp.wait()

    lanes = lax.iota(jnp.int32, _L)

    def splat(i, lc):
        bv = boxes_v[pl.ds(4 * i, _L)]
        cx = (bv[0] + bv[2]) // 2
        cy = (bv[1] + bv[3]) // 2
        y0 = jnp.clip(cy - _R, 0, _H - _L)
        dy = (lanes + (y0 - cy)).astype(jnp.float32)
        ey = jnp.exp(-(dy * dy))
        for r in range(2 * _R + 1):
            x = jnp.clip(cx + (r - _R), 0, _W - 1)
            g = ey if r == _R else _GX[r] * ey
            old = buf[lc, x, pl.ds(y0, _L)]
            buf[lc, x, pl.ds(y0, _L)] = jnp.maximum(old, g)

    def treemax(m):
        for sh in (1, 2, 4, 8):
            m = jnp.maximum(m, m.at[lanes ^ sh].get(mode="promise_in_bounds"))
        return m[0]

    def blk_body(b, carry):
        base = b * _L
        cls16 = cls_v[pl.ds(base, _L)]
        own = jnp.logical_and(cls16 >= c0, cls16 < c1)
        mv = jnp.where(own, lanes + 1, 0)

        def w_cond(st):
            return st[1] > 0

        def w_body(st):
            mv, mx = st
            j = mx - 1
            c = cls16.at[jnp.zeros((_L,), jnp.int32) + j].get(
                mode="promise_in_bounds"
            )[0]
            splat(base + j, c - c0)
            mv = jnp.where(lanes == j, 0, mv)
            return mv, treemax(mv)

        lax.while_loop(w_cond, w_body, (mv, treemax(mv)))
        return carry

    lax.fori_loop(0, _NBLK, blk_body, 0)

    pltpu.sync_copy(buf.at[pl.ds(0, 2)], out_hbm.at[pl.ds(c0, 2)])

    @pl.when(c1 - c0 == 3)
    def _():
        pltpu.sync_copy(buf.at[2], out_hbm.at[c0 + 2])


def kernel(heatmap, boxes, classes):
    boxes_flat = boxes.astype(jnp.int32).reshape(-1)
    cls_in = classes.astype(jnp.int32)

    mesh = plsc.VectorSubcoreMesh(
        core_axis_name="c", subcore_axis_name="s", num_cores=2, num_subcores=16
    )
    run = functools.partial(
        pl.kernel,
        out_type=jax.ShapeDtypeStruct((_C, _W, _H), jnp.float32),
        mesh=mesh,
        compiler_params=pltpu.CompilerParams(needs_layout_passes=False),
        scratch_types=[
            pltpu.VMEM((_MAXC, _W, _H), jnp.float32),
            pltpu.VMEM((4 * _NPAD,), jnp.int32),
            pltpu.VMEM((_NPAD,), jnp.int32),
            pltpu.SemaphoreType.DMA,
            pltpu.SemaphoreType.DMA,
        ],
    )(_splat_kernel)
    return run(heatmap, boxes_flat, cls_in)


# speedup vs baseline: 2.9468x; 1.0210x over previous
import functools
import math

import jax
import jax.numpy as jnp
from jax import lax
from jax.experimental import pallas as pl
from jax.experimental.pallas import tpu as pltpu
from jax.experimental.pallas import tpu_sc as plsc

_C, _W, _H = 80, 128, 128
_N = 1000
_NPAD = 1024
_NTILES = 32
_R = 4
_MAXC = 3
_L = 16
_NBLK = _NPAD // _L

_GX = [math.exp(-float((r - _R) ** 2)) for r in range(2 * _R + 1)]


def _splat_kernel(
    heat_hbm, boxes_hbm, classes_hbm, out_hbm, buf, boxes_v, cls_v, wl_v,
    sem_slab, sem_slab2, sem_boxes,
):
    wid = lax.axis_index("s") * 2 + lax.axis_index("c")
    c0 = (wid * _C) // _NTILES
    c1 = ((wid + 1) * _C) // _NTILES
    owns3 = (c1 - c0) == 3

    slab_cp = pltpu.make_async_copy(
        heat_hbm.at[pl.ds(c0, 2)], buf.at[pl.ds(0, 2)], sem_slab
    )
    slab_cp.start()
    slab2_cp = pltpu.make_async_copy(heat_hbm.at[c0 + 2], buf.at[2], sem_slab2)

    @pl.when(owns3)
    def _():
        slab2_cp.start()

    boxes_cp = pltpu.make_async_copy(
        boxes_hbm, boxes_v.at[pl.ds(0, 4 * _N)], sem_boxes
    )
    boxes_cp.start()
    sent = jnp.full((_L,), _C + 1, jnp.int32)
    cls_v[pl.ds(_N - 8, _L)] = sent
    cls_v[pl.ds(_N + 8, _L)] = sent
    pltpu.sync_copy(classes_hbm, cls_v.at[pl.ds(0, _N)])

    lanes = lax.iota(jnp.int32, _L)
    zeros16 = jnp.zeros((_L,), jnp.int32)

    def treemax(m):
        for sh in (1, 2, 4, 8):
            m = jnp.maximum(m, m.at[lanes ^ sh].get(mode="promise_in_bounds"))
        return m[0]

    def blk_body(b, pos):
        base = b * _L
        cls16 = cls_v[pl.ds(base, _L)]
        own = jnp.logical_and(cls16 >= c0, cls16 < c1)
        mv = jnp.where(own, lanes + 1, 0)

        def w_cond(st):
            return st[1] > 0

        def w_body(st):
            mv, mx, pos = st
            j = mx - 1
            c = cls16.at[zeros16 + j].get(mode="promise_in_bounds")[0]
            packed = (base + j) | ((c - c0) << 12)
            wl_v[pl.ds(pos, _L)] = zeros16 + packed
            mv = jnp.where(lanes == j, 0, mv)
            return mv, treemax(mv), pos + 1

        st = lax.while_loop(w_cond, w_body, (mv, treemax(mv), pos))
        return st[2]

    nwork = lax.fori_loop(0, _NBLK, blk_body, jnp.int32(0))

    boxes_cp.wait()
    slab_cp.wait()

    @pl.when(owns3)
    def _():
        slab2_cp.wait()

    def splat_body(k, carry):
        p = wl_v[pl.ds(k, _L)][0]
        i = p & 0xFFF
        lc = p >> 12
        bv = boxes_v[pl.ds(4 * i, _L)]
        cx = (bv[0] + bv[2]) // 2
        cy = (bv[1] + bv[3]) // 2
        y0 = jnp.clip(cy - _R, 0, _H - _L)
        dy = (lanes + (y0 - cy)).astype(jnp.float32)
        ey = jnp.exp(-(dy * dy))
        for r in range(2 * _R + 1):
            x = jnp.clip(cx + (r - _R), 0, _W - 1)
            g = ey if r == _R else _GX[r] * ey
            old = buf[lc, x, pl.ds(y0, _L)]
            buf[lc, x, pl.ds(y0, _L)] = jnp.maximum(old, g)
        return carry

    lax.fori_loop(0, nwork, splat_body, 0)

    pltpu.sync_copy(buf.at[pl.ds(0, 2)], out_hbm.at[pl.ds(c0, 2)])

    @pl.when(c1 - c0 == 3)
    def _():
        pltpu.sync_copy(buf.at[2], out_hbm.at[c0 + 2])


def kernel(heatmap, boxes, classes):
    boxes_flat = boxes.astype(jnp.int32).reshape(-1)
    cls_in = classes.astype(jnp.int32)

    mesh = plsc.VectorSubcoreMesh(
        core_axis_name="c", subcore_axis_name="s", num_cores=2, num_subcores=16
    )
    run = functools.partial(
        pl.kernel,
        out_type=jax.ShapeDtypeStruct((_C, _W, _H), jnp.float32),
        mesh=mesh,
        compiler_params=pltpu.CompilerParams(needs_layout_passes=False),
        scratch_types=[
            pltpu.VMEM((_MAXC, _W, _H), jnp.float32),
            pltpu.VMEM((4 * _NPAD,), jnp.int32),
            pltpu.VMEM((_NPAD,), jnp.int32),
            pltpu.VMEM((_NPAD + _L,), jnp.int32),
            pltpu.SemaphoreType.DMA,
            pltpu.SemaphoreType.DMA,
            pltpu.SemaphoreType.DMA,
        ],
    )(_splat_kernel)
    return run(heatmap, boxes_flat, cls_in)


# speedup vs baseline: 3.0285x; 1.0277x over previous
import functools
import math

import jax
import jax.numpy as jnp
from jax import lax
from jax.experimental import pallas as pl
from jax.experimental.pallas import tpu as pltpu
from jax.experimental.pallas import tpu_sc as plsc

_C, _W, _H = 80, 128, 128
_N = 1000
_NPAD = 1024
_NTILES = 32
_R = 4
_MAXC = 3
_L = 16
_NBLK = _NPAD // _L

_GX = [math.exp(-float((r - _R) ** 2)) for r in range(2 * _R + 1)]


def _splat_kernel(
    heat_hbm, boxes_hbm, classes_hbm, out_hbm, buf, boxes_v, cls_v, wl_v,
    sem_slab, sem_slab2, sem_boxes, sem_wb,
):
    wid = lax.axis_index("s") * 2 + lax.axis_index("c")
    c0 = (wid * _C) // _NTILES
    c1 = ((wid + 1) * _C) // _NTILES
    owns3 = (c1 - c0) == 3

    slab_cp = pltpu.make_async_copy(
        heat_hbm.at[pl.ds(c0, 2)], buf.at[pl.ds(0, 2)], sem_slab
    )
    slab_cp.start()
    slab2_cp = pltpu.make_async_copy(heat_hbm.at[c0 + 2], buf.at[2], sem_slab2)

    @pl.when(owns3)
    def _():
        slab2_cp.start()

    boxes_cp = pltpu.make_async_copy(
        boxes_hbm, boxes_v.at[pl.ds(0, 4 * _N)], sem_boxes
    )
    boxes_cp.start()
    sent = jnp.full((_L,), _C + 1, jnp.int32)
    cls_v[pl.ds(_N - 8, _L)] = sent
    cls_v[pl.ds(_N + 8, _L)] = sent
    pltpu.sync_copy(classes_hbm, cls_v.at[pl.ds(0, _N)])

    lanes = lax.iota(jnp.int32, _L)
    zeros16 = jnp.zeros((_L,), jnp.int32)

    def treemax(m):
        for sh in (1, 2, 4, 8):
            m = jnp.maximum(m, m.at[lanes ^ sh].get(mode="promise_in_bounds"))
        return m[0]

    def blk_body(b, posv):
        base = b * _L
        cls16 = cls_v[pl.ds(base, _L)]
        own = jnp.logical_and(cls16 >= c0, cls16 < c1)
        mv = jnp.where(own, lanes + 1, 0)

        def w_cond(st):
            return st[1] > 0

        def w_body(st):
            mv, mx, posv = st
            j = mx - 1
            c = cls16.at[zeros16 + j].get(mode="promise_in_bounds")[0]
            lc = c - c0
            pos = posv.at[zeros16 + lc].get(mode="promise_in_bounds")[0]
            wl_v[lc, pl.ds(pos, _L)] = zeros16 + (base + j)
            posv = posv + (lanes == lc).astype(jnp.int32)
            mv = jnp.where(lanes == j, 0, mv)
            return mv, treemax(mv), posv

        st = lax.while_loop(w_cond, w_body, (mv, treemax(mv), posv))
        return st[2]

    posv = lax.fori_loop(0, _NBLK, blk_body, zeros16)
    nch = c1 - c0

    boxes_cp.wait()
    slab_cp.wait()

    @pl.when(owns3)
    def _():
        slab2_cp.wait()

    def chan_body(lc, carry):
        cnt = posv.at[zeros16 + lc].get(mode="promise_in_bounds")[0]

        def splat_body(k, c2):
            i = wl_v[lc, pl.ds(k, _L)][0]
            bv = boxes_v[pl.ds(4 * i, _L)]
            cx = (bv[0] + bv[2]) // 2
            cy = (bv[1] + bv[3]) // 2
            y0 = jnp.clip(cy - _R, 0, _H - _L)
            dy = (lanes + (y0 - cy)).astype(jnp.float32)
            ey = jnp.exp(-(dy * dy))
            for r in range(2 * _R + 1):
                x = jnp.clip(cx + (r - _R), 0, _W - 1)
                g = ey if r == _R else _GX[r] * ey
                old = buf[lc, x, pl.ds(y0, _L)]
                buf[lc, x, pl.ds(y0, _L)] = jnp.maximum(old, g)
            return c2

        lax.fori_loop(0, cnt, splat_body, 0)
        pltpu.make_async_copy(buf.at[lc], out_hbm.at[c0 + lc], sem_wb).start()
        return carry

    lax.fori_loop(0, nch, chan_body, 0)

    waiter = pltpu.make_async_copy(buf.at[0], out_hbm.at[c0], sem_wb)

    def wait_body(lc, carry):
        waiter.wait()
        return carry

    lax.fori_loop(0, nch, wait_body, 0)


def kernel(heatmap, boxes, classes):
    boxes_flat = boxes.astype(jnp.int32).reshape(-1)
    cls_in = classes.astype(jnp.int32)

    mesh = plsc.VectorSubcoreMesh(
        core_axis_name="c", subcore_axis_name="s", num_cores=2, num_subcores=16
    )
    run = functools.partial(
        pl.kernel,
        out_type=jax.ShapeDtypeStruct((_C, _W, _H), jnp.float32),
        mesh=mesh,
        compiler_params=pltpu.CompilerParams(needs_layout_passes=False),
        scratch_types=[
            pltpu.VMEM((_MAXC, _W, _H), jnp.float32),
            pltpu.VMEM((4 * _NPAD,), jnp.int32),
            pltpu.VMEM((_NPAD,), jnp.int32),
            pltpu.VMEM((_MAXC, _NPAD + _L), jnp.int32),
            pltpu.SemaphoreType.DMA,
            pltpu.SemaphoreType.DMA,
            pltpu.SemaphoreType.DMA,
            pltpu.SemaphoreType.DMA,
        ],
    )(_splat_kernel)
    return run(heatmap, boxes_flat, cls_in)
